# baseline (speedup 1.0000x reference)
import functools
import math

import jax
import jax.numpy as jnp
from jax import lax
from jax.experimental import pallas as pl
from jax.experimental.pallas import tpu as pltpu
from jax.experimental.pallas import tpu_sc as plsc

D_MODEL = 64
SCALE = math.sqrt(D_MODEL)

NC = 2
NS = 16
NW = NC * NS
LANES = 16

CHUNK = 512


def _build(B):
    assert B % NW == 0
    b_per_w = B // NW
    assert b_per_w % CHUNK == 0
    n_chunks = b_per_w // CHUNK

    mesh = plsc.VectorSubcoreMesh(core_axis_name="c", subcore_axis_name="s")

    @functools.partial(
        pl.kernel,
        out_type=jax.ShapeDtypeStruct((B, D_MODEL), jnp.float32),
        mesh=mesh,
        scratch_types=[
            pltpu.VMEM((b_per_w,), jnp.int32),
            pltpu.VMEM((CHUNK, D_MODEL), jnp.float32),
            pltpu.SemaphoreType.DMA,
        ],
        compiler_params=pltpu.CompilerParams(use_tc_tiling_on_sc=False),
    )
    def emb(idx_hbm, table_hbm, out_hbm, idx_v, buf, sem):
        wid = lax.axis_index("s") * NC + lax.axis_index("c")
        base = wid * b_per_w
        pltpu.sync_copy(idx_hbm.at[pl.ds(base, b_per_w)], idx_v)

        def chunk_body(g, carry):
            pltpu.async_copy(
                table_hbm.at[idx_v.at[pl.ds(g * CHUNK, CHUNK)]], buf, sem
            ).wait()

            def scale_row(i, c):
                for j in range(D_MODEL // LANES):
                    sl = pl.ds(j * LANES, LANES)
                    buf[i, sl] = buf[i, sl] * SCALE
                return c

            lax.fori_loop(0, CHUNK, scale_row, 0)
            pltpu.sync_copy(buf, out_hbm.at[pl.ds(base + g * CHUNK, CHUNK)])
            return carry

        lax.fori_loop(0, n_chunks, chunk_body, 0)

    return emb


def kernel(x, table):
    orig_shape = x.shape
    xf = x.reshape(-1).astype(jnp.int32)
    out = _build(xf.shape[0])(xf, table)
    return out.reshape(orig_shape + (D_MODEL,))


# speedup vs baseline: 1.0241x; 1.0241x over previous
import functools
import math

import jax
import jax.numpy as jnp
from jax import lax
from jax.experimental import pallas as pl
from jax.experimental.pallas import tpu as pltpu
from jax.experimental.pallas import tpu_sc as plsc

D_MODEL = 64
SCALE = math.sqrt(D_MODEL)

NC = 2
NS = 16
NW = NC * NS
LANES = 16

CHUNK = 512


def _build(B):
    assert B % NW == 0
    b_per_w = B // NW
    assert b_per_w % CHUNK == 0
    n_chunks = b_per_w // CHUNK

    mesh = plsc.VectorSubcoreMesh(core_axis_name="c", subcore_axis_name="s")

    @functools.partial(
        pl.kernel,
        out_type=jax.ShapeDtypeStruct((B * D_MODEL,), jnp.float32),
        mesh=mesh,
        scratch_types=[
            pltpu.VMEM((b_per_w,), jnp.int32),
            pltpu.VMEM((CHUNK, D_MODEL), jnp.float32),
            pltpu.VMEM((CHUNK * D_MODEL,), jnp.float32),
            pltpu.SemaphoreType.DMA,
        ],
        compiler_params=pltpu.CompilerParams(use_tc_tiling_on_sc=False),
    )
    def emb(idx_hbm, table_hbm, out_hbm, idx_v, gbuf, wbuf, sem):
        wid = lax.axis_index("s") * NC + lax.axis_index("c")
        base = wid * b_per_w
        pltpu.sync_copy(idx_hbm.at[pl.ds(base, b_per_w)], idx_v)

        def chunk_body(g, carry):
            pltpu.async_copy(
                table_hbm.at[idx_v.at[pl.ds(g * CHUNK, CHUNK)]], gbuf, sem
            ).wait()

            def scale_row(i, c):
                for j in range(D_MODEL // LANES):
                    wbuf[pl.ds(i * D_MODEL + j * LANES, LANES)] = (
                        gbuf[i, pl.ds(j * LANES, LANES)] * SCALE
                    )
                return c

            lax.fori_loop(0, CHUNK, scale_row, 0)
            pltpu.sync_copy(
                wbuf,
                out_hbm.at[pl.ds((base + g * CHUNK) * D_MODEL, CHUNK * D_MODEL)],
            )
            return carry

        lax.fori_loop(0, n_chunks, chunk_body, 0)

    return emb


def kernel(x, table):
    orig_shape = x.shape
    xf = x.reshape(-1).astype(jnp.int32)
    out = _build(xf.shape[0])(xf, table)
    return out.reshape(orig_shape + (D_MODEL,))
